# baseline (speedup 1.0000x reference)
import functools

import jax
import jax.numpy as jnp
import numpy as np
from jax import lax
from jax.experimental import pallas as pl
from jax.experimental.pallas import tpu as pltpu
from jax.experimental.pallas import tpu_sc as plsc

HIDDEN = 1024
ROUTES = 8
BITS = 4
NGRAM = 4
ALPHA = 2 ** BITS
EMBED = 128
BATCH = 4
SEQ = 4096
TOKENS = BATCH * SEQ
NROWS_GATHER = TOKENS * ROUTES

_MCODE = np.zeros((ROUTES * BITS, ROUTES), np.float32)
_MSEL = np.zeros((ROUTES * BITS, ROUTES), np.float32)
for _r in range(ROUTES):
    for _b in range(BITS):
        _MCODE[_r * BITS + _b, _r] = float(2 ** _b)
        _MSEL[_r * BITS + _b, _r] = 1.0

_UREP = np.zeros((ROUTES, 128), np.float32)
_GMASK = np.zeros((16, 128), np.float32)
for _c in range(128):
    _UREP[_c % ROUTES, _c] = 1.0
    _GMASK[_c // ROUTES, _c] = 1.0


def _flatten16(V, u, gmask):
    F = jnp.dot(V, u, preferred_element_type=jnp.float32,
                precision=lax.Precision.HIGHEST)
    F3 = F.reshape(V.shape[0] // 16, 16, 128)
    return (F3 * gmask[None]).sum(axis=1)


def _route_body(x_ref, wr_ref, mcode_ref, msel_ref, urep_ref, gmask_ref,
                idx_ref, conf_ref):
    x = x_ref[0]
    w = wr_ref[...]
    logits = jnp.dot(x, w, preferred_element_type=jnp.float32)
    bitsf = (logits > 0.0).astype(jnp.float32)
    logq = jnp.log(jax.nn.sigmoid(jnp.abs(logits)))
    conf = jnp.exp(jnp.dot(logq, msel_ref[...],
                           preferred_element_type=jnp.float32,
                           precision=lax.Precision.HIGHEST))
    codes = jnp.dot(bitsf, mcode_ref[...],
                    preferred_element_type=jnp.float32,
                    precision=lax.Precision.HIGHEST)
    addr = codes
    for k in range(1, NGRAM):
        shifted = jnp.concatenate(
            [jnp.zeros((k, ROUTES), jnp.float32), codes[: SEQ - k]], axis=0)
        addr = addr + shifted * float(ALPHA ** k)
    off = lax.broadcasted_iota(jnp.int32, (SEQ, ROUTES), 1).astype(jnp.float32) \
        * float(ALPHA ** NGRAM)
    u, g = urep_ref[...], gmask_ref[...]
    idx_ref[...] = _flatten16(addr + off, u, g).astype(jnp.int32)
    conf_ref[...] = _flatten16(conf, u, g)


def _route_call(x, W_route):
    return pl.pallas_call(
        _route_body,
        grid=(BATCH,),
        in_specs=[
            pl.BlockSpec((1, SEQ, HIDDEN), lambda b: (b, 0, 0)),
            pl.BlockSpec((HIDDEN, ROUTES * BITS), lambda b: (0, 0)),
            pl.BlockSpec((ROUTES * BITS, ROUTES), lambda b: (0, 0)),
            pl.BlockSpec((ROUTES * BITS, ROUTES), lambda b: (0, 0)),
            pl.BlockSpec((ROUTES, 128), lambda b: (0, 0)),
            pl.BlockSpec((16, 128), lambda b: (0, 0)),
        ],
        out_specs=[
            pl.BlockSpec((SEQ // 16, 128), lambda b: (b, 0)),
            pl.BlockSpec((SEQ // 16, 128), lambda b: (b, 0)),
        ],
        out_shape=[
            jax.ShapeDtypeStruct((NROWS_GATHER // 128, 128), jnp.int32),
            jax.ShapeDtypeStruct((NROWS_GATHER // 128, 128), jnp.float32),
        ],
    )(x, W_route, jnp.asarray(_MCODE), jnp.asarray(_MSEL),
      jnp.asarray(_UREP), jnp.asarray(_GMASK))


NW = 32
TOK_PER_W = TOKENS // NW
RPC = 128
CH = RPC // ROUTES
NCH = TOK_PER_W // CH
LANES = 16


def _sc_pool_body(table_hbm, idx_hbm, conf_hbm, out_hbm,
                  idx_v, conf_v, rows0, rows1, out_v, sem0, sem1):
    wid = lax.axis_index("s") * 2 + lax.axis_index("c")
    pltpu.sync_copy(idx_hbm.at[pl.ds(wid * NCH, NCH)], idx_v)
    pltpu.sync_copy(conf_hbm.at[pl.ds(wid * NCH * RPC, NCH * RPC)], conf_v)
    pltpu.async_copy(table_hbm.at[idx_v.at[0]], rows0, sem0)

    def chunk_compute(cb, buf):
        cbase = cb * RPC

        def tok(i, carry):
            j0 = i * ROUTES
            accs = [jnp.zeros((LANES,), jnp.float32) for _ in range(8)]
            for r in range(ROUTES):
                j = j0 + r
                cvec = plsc.load_gather(
                    conf_v, [jnp.full((LANES,), cbase, jnp.int32) + j])
                for k in range(8):
                    accs[k] = accs[k] + cvec * buf[j, pl.ds(k * LANES, LANES)]
            for k in range(8):
                out_v[i, pl.ds(k * LANES, LANES)] = accs[k]
            return carry

        lax.fori_loop(0, CH, tok, 0)

    def step(c, carry):
        c0 = 2 * c
        pltpu.make_async_copy(table_hbm.at[idx_v.at[0]], rows0, sem0).wait()
        pltpu.async_copy(table_hbm.at[idx_v.at[c0 + 1]], rows1, sem1)
        chunk_compute(c0, rows0)
        pltpu.sync_copy(out_v, out_hbm.at[pl.ds(wid * TOK_PER_W + c0 * CH, CH)])
        pltpu.make_async_copy(table_hbm.at[idx_v.at[0]], rows1, sem1).wait()

        @pl.when(c < NCH // 2 - 1)
        def _():
            pltpu.async_copy(table_hbm.at[idx_v.at[c0 + 2]], rows0, sem0)

        chunk_compute(c0 + 1, rows1)
        pltpu.sync_copy(out_v,
                        out_hbm.at[pl.ds(wid * TOK_PER_W + (c0 + 1) * CH, CH)])
        return carry

    lax.fori_loop(0, NCH // 2, step, 0)


@functools.lru_cache(maxsize=1)
def _get_sc_pool():
    return functools.partial(
        pl.kernel,
        mesh=plsc.VectorSubcoreMesh(core_axis_name="c", subcore_axis_name="s"),
        compiler_params=pltpu.CompilerParams(needs_layout_passes=False),
        out_type=jax.ShapeDtypeStruct((TOKENS, EMBED), jnp.float32),
        scratch_types=[
            pltpu.VMEM((NCH, RPC), jnp.int32),
            pltpu.VMEM((NCH * RPC,), jnp.float32),
            pltpu.VMEM((RPC, EMBED), jnp.float32),
            pltpu.VMEM((RPC, EMBED), jnp.float32),
            pltpu.VMEM((CH, EMBED), jnp.float32),
            pltpu.SemaphoreType.DMA,
            pltpu.SemaphoreType.DMA,
        ],
    )(_sc_pool_body)


_BT = 512


def _out_body(p_ref, w_ref, o_ref):
    o_ref[...] = jnp.dot(p_ref[...], w_ref[...],
                         preferred_element_type=jnp.float32)


def _out_call(pooled, W_out):
    return pl.pallas_call(
        _out_body,
        grid=(TOKENS // _BT,),
        in_specs=[
            pl.BlockSpec((_BT, EMBED), lambda i: (i, 0)),
            pl.BlockSpec((EMBED, HIDDEN), lambda i: (0, 0)),
        ],
        out_specs=pl.BlockSpec((_BT, HIDDEN), lambda i: (i, 0)),
        out_shape=jax.ShapeDtypeStruct((TOKENS, HIDDEN), jnp.float32),
    )(pooled, W_out)


def kernel(x, W_route, table, W_out):
    B, T, _ = x.shape
    idx2, conf2 = _route_call(x, W_route)
    pooled = _get_sc_pool()(table, idx2, conf2.reshape(-1))
    out = _out_call(pooled, W_out)
    return out.reshape(B, T, HIDDEN)


# speedup vs baseline: 1.2365x; 1.2365x over previous
import functools

import jax
import jax.numpy as jnp
import numpy as np
from jax import lax
from jax.experimental import pallas as pl
from jax.experimental.pallas import tpu as pltpu
from jax.experimental.pallas import tpu_sc as plsc

HIDDEN = 1024
ROUTES = 8
BITS = 4
NGRAM = 4
ALPHA = 2 ** BITS
EMBED = 128
BATCH = 4
SEQ = 4096
TOKENS = BATCH * SEQ
NROWS_GATHER = TOKENS * ROUTES

_MCODE = np.zeros((ROUTES * BITS, ROUTES), np.float32)
_MSEL = np.zeros((ROUTES * BITS, ROUTES), np.float32)
for _r in range(ROUTES):
    for _b in range(BITS):
        _MCODE[_r * BITS + _b, _r] = float(2 ** _b)
        _MSEL[_r * BITS + _b, _r] = 1.0

_UREP = np.zeros((ROUTES, 128), np.float32)
_GMASK = np.zeros((16, 128), np.float32)
for _c in range(128):
    _UREP[_c % ROUTES, _c] = 1.0
    _GMASK[_c // ROUTES, _c] = 1.0


def _flatten16(V, u, gmask):
    F = jnp.dot(V, u, preferred_element_type=jnp.float32)
    F3 = F.reshape(V.shape[0] // 16, 16, 128)
    return (F3 * gmask[None]).sum(axis=1)


def _route_body(x_ref, wr_ref, mcode_ref, msel_ref, urep_ref, gmask_ref,
                idx_ref, conf_ref):
    x = x_ref[0]
    w = wr_ref[...]
    logits = jnp.dot(x, w, preferred_element_type=jnp.float32)
    bitsf = (logits > 0.0).astype(jnp.float32)
    logq = jnp.log(jax.nn.sigmoid(jnp.abs(logits)))
    conf = jnp.exp(jnp.dot(logq, msel_ref[...],
                           preferred_element_type=jnp.float32))
    codes = jnp.dot(bitsf, mcode_ref[...],
                    preferred_element_type=jnp.float32)

    def shift(v, k):
        return jnp.concatenate(
            [jnp.zeros((k, ROUTES), jnp.float32), v[: SEQ - k]], axis=0)

    a01 = codes + shift(codes, 1) * float(ALPHA)
    a23 = shift(codes, 2) + shift(codes, 3) * float(ALPHA)
    u, g = urep_ref[...], gmask_ref[...]
    f01 = _flatten16(a01, u, g)
    f23 = _flatten16(a23, u, g)
    roff = (lax.broadcasted_iota(jnp.int32, (SEQ // 16, 128), 1) % ROUTES) \
        * (ALPHA ** NGRAM)
    idx_ref[...] = f01.astype(jnp.int32) \
        + f23.astype(jnp.int32) * (ALPHA * ALPHA) + roff
    conf_ref[...] = _flatten16(conf, u, g)


def _route_call(x, W_route):
    return pl.pallas_call(
        _route_body,
        grid=(BATCH,),
        in_specs=[
            pl.BlockSpec((1, SEQ, HIDDEN), lambda b: (b, 0, 0)),
            pl.BlockSpec((HIDDEN, ROUTES * BITS), lambda b: (0, 0)),
            pl.BlockSpec((ROUTES * BITS, ROUTES), lambda b: (0, 0)),
            pl.BlockSpec((ROUTES * BITS, ROUTES), lambda b: (0, 0)),
            pl.BlockSpec((ROUTES, 128), lambda b: (0, 0)),
            pl.BlockSpec((16, 128), lambda b: (0, 0)),
        ],
        out_specs=[
            pl.BlockSpec((SEQ // 16, 128), lambda b: (b, 0)),
            pl.BlockSpec((SEQ // 16, 128), lambda b: (b, 0)),
        ],
        out_shape=[
            jax.ShapeDtypeStruct((NROWS_GATHER // 128, 128), jnp.int32),
            jax.ShapeDtypeStruct((NROWS_GATHER // 128, 128), jnp.float32),
        ],
    )(x, W_route, jnp.asarray(_MCODE), jnp.asarray(_MSEL),
      jnp.asarray(_UREP), jnp.asarray(_GMASK))


NW = 32
TOK_PER_W = TOKENS // NW
RPC = 128
CH = RPC // ROUTES
NCH = TOK_PER_W // CH
LANES = 16


def _sc_pool_body(table_hbm, idx_hbm, conf_hbm, out_hbm,
                  idx_v, conf_v, rows0, rows1, out_v, sem0, sem1):
    wid = lax.axis_index("s") * 2 + lax.axis_index("c")
    pltpu.sync_copy(idx_hbm.at[pl.ds(wid * NCH, NCH)], idx_v)
    pltpu.sync_copy(conf_hbm.at[pl.ds(wid * NCH * RPC, NCH * RPC)], conf_v)
    pltpu.async_copy(table_hbm.at[idx_v.at[0]], rows0, sem0)

    def chunk_compute(cb, buf):
        cbase = cb * RPC

        def tok(i, carry):
            j0 = i * ROUTES
            accs = [jnp.zeros((LANES,), jnp.float32) for _ in range(8)]
            for r in range(ROUTES):
                j = j0 + r
                cvec = plsc.load_gather(
                    conf_v, [jnp.full((LANES,), cbase, jnp.int32) + j])
                for k in range(8):
                    accs[k] = accs[k] + cvec * buf[j, pl.ds(k * LANES, LANES)]
            for k in range(8):
                out_v[i, pl.ds(k * LANES, LANES)] = accs[k]
            return carry

        lax.fori_loop(0, CH, tok, 0)

    def step(c, carry):
        c0 = 2 * c
        pltpu.make_async_copy(table_hbm.at[idx_v.at[0]], rows0, sem0).wait()
        pltpu.async_copy(table_hbm.at[idx_v.at[c0 + 1]], rows1, sem1)
        chunk_compute(c0, rows0)
        pltpu.sync_copy(out_v, out_hbm.at[pl.ds(wid * TOK_PER_W + c0 * CH, CH)])
        pltpu.make_async_copy(table_hbm.at[idx_v.at[0]], rows1, sem1).wait()

        @pl.when(c < NCH // 2 - 1)
        def _():
            pltpu.async_copy(table_hbm.at[idx_v.at[c0 + 2]], rows0, sem0)

        chunk_compute(c0 + 1, rows1)
        pltpu.sync_copy(out_v,
                        out_hbm.at[pl.ds(wid * TOK_PER_W + (c0 + 1) * CH, CH)])
        return carry

    lax.fori_loop(0, NCH // 2, step, 0)


@functools.lru_cache(maxsize=1)
def _get_sc_pool():
    return functools.partial(
        pl.kernel,
        mesh=plsc.VectorSubcoreMesh(core_axis_name="c", subcore_axis_name="s"),
        compiler_params=pltpu.CompilerParams(needs_layout_passes=False),
        out_type=jax.ShapeDtypeStruct((TOKENS, EMBED), jnp.float32),
        scratch_types=[
            pltpu.VMEM((NCH, RPC), jnp.int32),
            pltpu.VMEM((NCH * RPC,), jnp.float32),
            pltpu.VMEM((RPC, EMBED), jnp.float32),
            pltpu.VMEM((RPC, EMBED), jnp.float32),
            pltpu.VMEM((CH, EMBED), jnp.float32),
            pltpu.SemaphoreType.DMA,
            pltpu.SemaphoreType.DMA,
        ],
    )(_sc_pool_body)


_BT = 512


def _out_body(p_ref, w_ref, o_ref):
    o_ref[...] = jnp.dot(p_ref[...], w_ref[...],
                         preferred_element_type=jnp.float32)


def _out_call(pooled, W_out):
    return pl.pallas_call(
        _out_body,
        grid=(TOKENS // _BT,),
        in_specs=[
            pl.BlockSpec((_BT, EMBED), lambda i: (i, 0)),
            pl.BlockSpec((EMBED, HIDDEN), lambda i: (0, 0)),
        ],
        out_specs=pl.BlockSpec((_BT, HIDDEN), lambda i: (i, 0)),
        out_shape=jax.ShapeDtypeStruct((TOKENS, HIDDEN), jnp.float32),
    )(pooled, W_out)


def kernel(x, W_route, table, W_out):
    B, T, _ = x.shape
    idx2, conf2 = _route_call(x, W_route)
    pooled = _get_sc_pool()(table, idx2, conf2.reshape(-1))
    out = _out_call(pooled, W_out)
    return out.reshape(B, T, HIDDEN)


# speedup vs baseline: 1.3867x; 1.1215x over previous
import functools

import jax
import jax.numpy as jnp
import numpy as np
from jax import lax
from jax.experimental import pallas as pl
from jax.experimental.pallas import tpu as pltpu
from jax.experimental.pallas import tpu_sc as plsc

HIDDEN = 1024
ROUTES = 8
BITS = 4
NGRAM = 4
ALPHA = 2 ** BITS
EMBED = 128
BATCH = 4
SEQ = 4096
TOKENS = BATCH * SEQ
NROWS_GATHER = TOKENS * ROUTES

_MCODE = np.zeros((ROUTES * BITS, ROUTES), np.float32)
_MSEL = np.zeros((ROUTES * BITS, ROUTES), np.float32)
for _r in range(ROUTES):
    for _b in range(BITS):
        _MCODE[_r * BITS + _b, _r] = float(2 ** _b)
        _MSEL[_r * BITS + _b, _r] = 1.0

_UREP = np.zeros((ROUTES, 128), np.float32)
_GMASK = np.zeros((16, 128), np.float32)
for _c in range(128):
    _UREP[_c % ROUTES, _c] = 1.0
    _GMASK[_c // ROUTES, _c] = 1.0


def _flatten16(V, u, gmask):
    F = jnp.dot(V, u, preferred_element_type=jnp.float32)
    F3 = F.reshape(V.shape[0] // 16, 16, 128)
    return (F3 * gmask[None]).sum(axis=1)


def _route_body(x_ref, wr_ref, mcode_ref, msel_ref, urep_ref, gmask_ref,
                idx_ref, conf_ref):
    x = x_ref[0]
    w = wr_ref[...]
    logits = jnp.dot(x, w, preferred_element_type=jnp.float32)
    bitsf = (logits > 0.0).astype(jnp.float32)
    logq = jnp.log(jax.nn.sigmoid(jnp.abs(logits)))
    conf = jnp.exp(jnp.dot(logq, msel_ref[...],
                           preferred_element_type=jnp.float32))
    codes = jnp.dot(bitsf, mcode_ref[...],
                    preferred_element_type=jnp.float32)

    def shift(v, k):
        return jnp.concatenate(
            [jnp.zeros((k, ROUTES), jnp.float32), v[: SEQ - k]], axis=0)

    a01 = codes + shift(codes, 1) * float(ALPHA)
    a23 = shift(codes, 2) + shift(codes, 3) * float(ALPHA)
    u, g = urep_ref[...], gmask_ref[...]
    f01 = _flatten16(a01, u, g)
    f23 = _flatten16(a23, u, g)
    roff = (lax.broadcasted_iota(jnp.int32, (SEQ // 16, 128), 1) % ROUTES) \
        * (ALPHA ** NGRAM)
    idx_ref[...] = f01.astype(jnp.int32) \
        + f23.astype(jnp.int32) * (ALPHA * ALPHA) + roff
    conf_ref[...] = _flatten16(conf, u, g)


HB = BATCH // 2
HTOK = HB * SEQ
HROWS = HTOK * ROUTES // 128


def _route_call_half(x, W_route, half):
    return pl.pallas_call(
        _route_body,
        grid=(HB,),
        in_specs=[
            pl.BlockSpec((1, SEQ, HIDDEN), lambda b, h=half: (b + HB * h, 0, 0)),
            pl.BlockSpec((HIDDEN, ROUTES * BITS), lambda b: (0, 0)),
            pl.BlockSpec((ROUTES * BITS, ROUTES), lambda b: (0, 0)),
            pl.BlockSpec((ROUTES * BITS, ROUTES), lambda b: (0, 0)),
            pl.BlockSpec((ROUTES, 128), lambda b: (0, 0)),
            pl.BlockSpec((16, 128), lambda b: (0, 0)),
        ],
        out_specs=[
            pl.BlockSpec((SEQ // 16, 128), lambda b: (b, 0)),
            pl.BlockSpec((SEQ // 16, 128), lambda b: (b, 0)),
        ],
        out_shape=[
            jax.ShapeDtypeStruct((HROWS, 128), jnp.int32),
            jax.ShapeDtypeStruct((HROWS, 128), jnp.float32),
        ],
    )(x, W_route, jnp.asarray(_MCODE), jnp.asarray(_MSEL),
      jnp.asarray(_UREP), jnp.asarray(_GMASK))


NW = 32
TOK_PER_W = TOKENS // NW
RPC = 128
CH = RPC // ROUTES
NCH = TOK_PER_W // CH
LANES = 16


def _make_sc_body(nch):
    tok_per_w = nch * CH

    def body(table_hbm, idx_hbm, conf_hbm, out_hbm,
             idx_v, conf_v, rows0, rows1, out_v, sem0, sem1):
        wid = lax.axis_index("s") * 2 + lax.axis_index("c")
        pltpu.sync_copy(idx_hbm.at[pl.ds(wid * nch, nch)], idx_v)
        pltpu.sync_copy(conf_hbm.at[pl.ds(wid * nch * RPC, nch * RPC)], conf_v)
        pltpu.async_copy(table_hbm.at[idx_v.at[0]], rows0, sem0)

        def chunk_compute(cb, buf):
            cbase = cb * RPC

            def tok(i, carry):
                j0 = i * ROUTES
                accs = [jnp.zeros((LANES,), jnp.float32) for _ in range(8)]
                for r in range(ROUTES):
                    j = j0 + r
                    cvec = plsc.load_gather(
                        conf_v, [jnp.full((LANES,), cbase, jnp.int32) + j])
                    for k in range(8):
                        accs[k] = accs[k] + cvec * buf[j, pl.ds(k * LANES, LANES)]
                for k in range(8):
                    out_v[i, pl.ds(k * LANES, LANES)] = accs[k]
                return carry

            lax.fori_loop(0, CH, tok, 0)

        def step(c, carry):
            c0 = 2 * c
            pltpu.make_async_copy(table_hbm.at[idx_v.at[0]], rows0, sem0).wait()
            pltpu.async_copy(table_hbm.at[idx_v.at[c0 + 1]], rows1, sem1)
            chunk_compute(c0, rows0)
            pltpu.sync_copy(out_v,
                            out_hbm.at[pl.ds(wid * tok_per_w + c0 * CH, CH)])
            pltpu.make_async_copy(table_hbm.at[idx_v.at[0]], rows1, sem1).wait()

            @pl.when(c < nch // 2 - 1)
            def _():
                pltpu.async_copy(table_hbm.at[idx_v.at[c0 + 2]], rows0, sem0)

            chunk_compute(c0 + 1, rows1)
            pltpu.sync_copy(out_v,
                            out_hbm.at[pl.ds(wid * tok_per_w + (c0 + 1) * CH,
                                             CH)])
            return carry

        lax.fori_loop(0, nch // 2, step, 0)

    return body


@functools.lru_cache(maxsize=2)
def _get_sc_pool(nch):
    ntok = nch * CH * NW
    return functools.partial(
        pl.kernel,
        mesh=plsc.VectorSubcoreMesh(core_axis_name="c", subcore_axis_name="s"),
        compiler_params=pltpu.CompilerParams(needs_layout_passes=False),
        out_type=jax.ShapeDtypeStruct((ntok, EMBED), jnp.float32),
        scratch_types=[
            pltpu.VMEM((nch, RPC), jnp.int32),
            pltpu.VMEM((nch * RPC,), jnp.float32),
            pltpu.VMEM((RPC, EMBED), jnp.float32),
            pltpu.VMEM((RPC, EMBED), jnp.float32),
            pltpu.VMEM((CH, EMBED), jnp.float32),
            pltpu.SemaphoreType.DMA,
            pltpu.SemaphoreType.DMA,
        ],
    )(_make_sc_body(nch))


_BT = 512


def _out_body(p_ref, w_ref, o_ref):
    o_ref[...] = jnp.dot(p_ref[...], w_ref[...],
                         preferred_element_type=jnp.float32)


def _out_call_1(pooled, W_out):
    return pl.pallas_call(
        _out_body,
        grid=(HTOK // _BT,),
        in_specs=[
            pl.BlockSpec((_BT, EMBED), lambda i: (i, 0)),
            pl.BlockSpec((EMBED, HIDDEN), lambda i: (0, 0)),
        ],
        out_specs=pl.BlockSpec((_BT, HIDDEN), lambda i: (i, 0)),
        out_shape=jax.ShapeDtypeStruct((TOKENS, HIDDEN), jnp.float32),
    )(pooled, W_out)


def _out_body2(o_ref, p_ref, w_ref, out_ref):
    del o_ref
    out_ref[...] = jnp.dot(p_ref[...], w_ref[...],
                           preferred_element_type=jnp.float32)


def _out_call_2(o1, pooled, W_out):
    return pl.pallas_call(
        _out_body2,
        grid=(HTOK // _BT,),
        in_specs=[
            pl.BlockSpec(memory_space=pl.ANY),
            pl.BlockSpec((_BT, EMBED), lambda i: (i, 0)),
            pl.BlockSpec((EMBED, HIDDEN), lambda i: (0, 0)),
        ],
        out_specs=pl.BlockSpec((_BT, HIDDEN), lambda i: (i + HTOK // _BT, 0)),
        out_shape=jax.ShapeDtypeStruct((TOKENS, HIDDEN), jnp.float32),
        input_output_aliases={0: 0},
    )(o1, pooled, W_out)


def kernel(x, W_route, table, W_out):
    B, T, _ = x.shape
    idx_a, conf_a = _route_call_half(x, W_route, 0)
    idx_b, conf_b = _route_call_half(x, W_route, 1)
    p1 = _get_sc_pool(HROWS // NW)(table, idx_a, conf_a.reshape(-1))
    p2 = _get_sc_pool(HROWS // NW)(table, idx_b, conf_b.reshape(-1))
    o1 = _out_call_1(p1, W_out)
    out = _out_call_2(o1, p2, W_out)
    return out.reshape(B, T, HIDDEN)
